# baseline (speedup 1.0000x reference)
import functools

import jax
import jax.numpy as jnp
from jax import lax
from jax.experimental import pallas as pl
from jax.experimental.pallas import tpu as pltpu
from jax.experimental.pallas import tpu_sc as plsc

_H = 4
_QD = 1024
_SKD = 512
_NSUB = 256
_K = 16
_DIM = 768
_SEQ = 2048
_HQD = _H * _QD

_TB = 256
_GRID = _SEQ // _TB

_STAIR = [(i, j) for i in range(_K) for j in range(_K) if (i + 1) * (j + 1) <= _K]

_NEG_INF = float("-inf")
_BIG_I = 1 << 30


_QTILE = 1024

def _mm_body(w_ref, x_ref, b_ref, s_ref, ss_ref):
    xq = lax.dot_general(
        w_ref[...], x_ref[...],
        ((( 0,), (1,)), ((), ())),
        preferred_element_type=jnp.float32,
    ) + b_ref[...]
    s_ref[...] = jnp.sum(xq, axis=1, keepdims=True)
    ss_ref[...] = jnp.sum(xq * xq, axis=1, keepdims=True)


_matmul_stats = pl.pallas_call(
    _mm_body,
    grid=(_HQD // _QTILE,),
    in_specs=[
        pl.BlockSpec((_DIM, _QTILE), lambda i: (0, i)),
        pl.BlockSpec((_SEQ, _DIM), lambda i: (0, 0)),
        pl.BlockSpec((_QTILE, 1), lambda i: (i, 0)),
    ],
    out_specs=[
        pl.BlockSpec((_QTILE, 1), lambda i: (i, 0)),
        pl.BlockSpec((_QTILE, 1), lambda i: (i, 0)),
    ],
    out_shape=[
        jax.ShapeDtypeStruct((_HQD, 1), jnp.float32),
        jax.ShapeDtypeStruct((_HQD, 1), jnp.float32),
    ],
)


def _topk16_vi(s_lo, s_hi):
    iota = lax.broadcasted_iota(jnp.int32, s_lo.shape, 0)
    work = s_lo
    vrows, irows = [], []
    for _ in range(_K):
        m = jnp.max(work, axis=0, keepdims=True)
        am = jnp.min(jnp.where(work == m, iota, _BIG_I), axis=0, keepdims=True)
        sel = iota == am
        v = jnp.min(jnp.where(sel, s_hi, jnp.inf), axis=0, keepdims=True)
        vrows.append(v)
        irows.append(am)
        work = jnp.where(sel, _NEG_INF, work)
    return jnp.concatenate(vrows, axis=0), jnp.concatenate(irows, axis=0)


def _topk16_payload(s, gi):
    iota = lax.broadcasted_iota(jnp.int32, s.shape, 0)
    work = s
    vrows, grows = [], []
    for _ in range(_K):
        m = jnp.max(work, axis=0, keepdims=True)
        am = jnp.min(jnp.where(work == m, iota, _BIG_I), axis=0, keepdims=True)
        sel = iota == am
        g = jnp.min(jnp.where(sel, gi, _BIG_I), axis=0, keepdims=True)
        vrows.append(m)
        grows.append(g)
        work = jnp.where(sel, _NEG_INF, work)
    return jnp.concatenate(vrows, axis=0), jnp.concatenate(grows, axis=0)


def _st_body(wq_ref, x_ref, bq_ref, s_ref, ss_ref, g_ref, b_ref, c1_ref,
             c2_ref, w_ref, gi_ref):
    ntok = jnp.float32(_SEQ)
    mean = s_ref[...] / ntok
    msq = ss_ref[...] / ntok
    var = msq - mean * mean
    scale = g_ref[...] * lax.rsqrt(var + 1e-5)
    shift = b_ref[...] - mean * scale
    qT = lax.dot_general(
        wq_ref[...], x_ref[...],
        ((( 0,), (1,)), ((), ())),
        preferred_element_type=jnp.float32,
    ) + bq_ref[...]
    qn = qT * scale + shift

    for h in range(_H):
        q1 = qn[h * _QD : h * _QD + _SKD, :]
        q2 = qn[h * _QD + _SKD : (h + 1) * _QD, :]
        s1_lo = jnp.dot(c1_ref[h], q1, preferred_element_type=jnp.float32)
        s2_lo = jnp.dot(c2_ref[h], q2, preferred_element_type=jnp.float32)
        s1_hi = jnp.dot(c1_ref[h], q1, preferred_element_type=jnp.float32,
                        precision=lax.Precision.HIGHEST)
        s2_hi = jnp.dot(c2_ref[h], q2, preferred_element_type=jnp.float32,
                        precision=lax.Precision.HIGHEST)
        v1, i1 = _topk16_vi(s1_lo, s1_hi)
        v2, i2 = _topk16_vi(s2_lo, s2_hi)
        cand_rows, gi_rows = [], []
        for i in range(_K):
            cnt = sum(1 for ii, _ in _STAIR if ii == i)
            cand_rows.append(v1[i : i + 1, :] + v2[:cnt, :])
            gi_rows.append(i1[i : i + 1, :] * _NSUB + i2[:cnt, :])
        cand = jnp.concatenate(cand_rows, axis=0)
        gcand = jnp.concatenate(gi_rows, axis=0)
        tv, tg = _topk16_payload(cand, gcand)
        m = jnp.max(tv, axis=0, keepdims=True)
        e = jnp.exp(tv - m)
        wts = e / jnp.sum(e, axis=0, keepdims=True)
        w_ref[h * _K : (h + 1) * _K, :] = wts
        gi_ref[h * _K : (h + 1) * _K, :] = tg


_scores_topk = pl.pallas_call(
    _st_body,
    grid=(_GRID,),
    in_specs=[
        pl.BlockSpec((_DIM, _HQD), lambda i: (0, 0)),
        pl.BlockSpec((_TB, _DIM), lambda i: (i, 0)),
        pl.BlockSpec((_HQD, 1), lambda i: (0, 0)),
        pl.BlockSpec((_HQD, 1), lambda i: (0, 0)),
        pl.BlockSpec((_HQD, 1), lambda i: (0, 0)),
        pl.BlockSpec((_HQD, 1), lambda i: (0, 0)),
        pl.BlockSpec((_HQD, 1), lambda i: (0, 0)),
        pl.BlockSpec((_H, _NSUB, _SKD), lambda i: (0, 0, 0)),
        pl.BlockSpec((_H, _NSUB, _SKD), lambda i: (0, 0, 0)),
    ],
    out_specs=[
        pl.BlockSpec((_H * _K, _TB), lambda i: (0, i)),
        pl.BlockSpec((_H * _K, _TB), lambda i: (0, i)),
    ],
    out_shape=[
        jax.ShapeDtypeStruct((_H * _K, _SEQ), jnp.float32),
        jax.ShapeDtypeStruct((_H * _K, _SEQ), jnp.int32),
    ],
)


_NC = 2
_NS = 16
_NW = _NC * _NS
_TPW = _SEQ // _NW
_R = _H * _K
_CG = 8
_NCG = _DIM // (16 * _CG)


def _sc_body(idx_hbm, w_hbm, values_hbm, out_hbm, idx_v, w_v, rows_v, outb_v,
             sem0, sem1, semo0, semo1):
    wid = lax.axis_index("s") * _NC + lax.axis_index("c")
    base = wid * _TPW
    pltpu.sync_copy(idx_hbm.at[pl.ds(base, _TPW)], idx_v)
    pltpu.sync_copy(w_hbm.at[pl.ds(base, _TPW)], w_v)
    sems = (sem0, sem1)
    semos = (semo0, semo1)

    def _issue(t, buf):
        pltpu.async_copy(values_hbm.at[idx_v.at[t]], rows_v.at[buf], sems[buf])

    _issue(0, 0)
    _issue(1, 1)

    def _token(t, buf):
        pltpu.make_async_copy(
            values_hbm.at[idx_v.at[t]], rows_v.at[buf], sems[buf]
        ).wait()

        @pl.when(t >= 2)
        def _():
            pltpu.make_async_copy(
                outb_v.at[buf], out_hbm.at[base + t - 2], semos[buf]
            ).wait()

        for cg in range(_NCG):
            def rgbody(rg, accs, _cg=cg):
                wvec = w_v[t, pl.ds(rg * 16, 16)]
                new = list(accs)
                for j in range(16):
                    wsc = wvec[j]
                    r = rg * 16 + j
                    for k in range(_CG):
                        new[k] = new[k] + (
                            rows_v[buf, r, pl.ds((_cg * _CG + k) * 16, 16)] * wsc
                        )
                return tuple(new)
            accs = lax.fori_loop(
                0, _R // 16, rgbody,
                tuple(jnp.zeros((16,), jnp.float32) for _ in range(_CG)),
            )
            for j in range(_CG):
                outb_v[buf, pl.ds((cg * _CG + j) * 16, 16)] = accs[j]
        pltpu.async_copy(outb_v.at[buf], out_hbm.at[base + t], semos[buf])

        @pl.when(t + 2 < _TPW)
        def _():
            _issue(t + 2, buf)

    def _outer(ti, carry):
        _token(ti * 2, 0)
        _token(ti * 2 + 1, 1)
        return carry

    lax.fori_loop(0, _TPW // 2, _outer, jnp.int32(0))
    pltpu.make_async_copy(
        outb_v.at[0], out_hbm.at[base + _TPW - 2], semo0).wait()
    pltpu.make_async_copy(
        outb_v.at[1], out_hbm.at[base + _TPW - 1], semo1).wait()


@functools.cache
def _sc_gather():
    return functools.partial(
        pl.kernel,
        mesh=plsc.VectorSubcoreMesh(core_axis_name="c", subcore_axis_name="s"),
        out_type=jax.ShapeDtypeStruct((_SEQ, _DIM), jnp.float32),
        scratch_types=[
            pltpu.VMEM((_TPW, _R), jnp.int32),
            pltpu.VMEM((_TPW, _R), jnp.float32),
            pltpu.VMEM((2, _R, _DIM), jnp.float32),
            pltpu.VMEM((2, _DIM), jnp.float32),
            pltpu.SemaphoreType.DMA,
            pltpu.SemaphoreType.DMA,
            pltpu.SemaphoreType.DMA,
            pltpu.SemaphoreType.DMA,
        ],
    )(_sc_body)


def kernel(x, W_q, b_q, bn_gamma, bn_beta, c1, c2, values):
    bs, seq, d = x.shape
    x_flat = x.reshape(-1, d)
    bq_col = b_q.reshape(-1, 1)
    psum, psumsq = _matmul_stats(W_q, x_flat, bq_col)
    wtsT, gidxT = _scores_topk(
        W_q, x_flat, bq_col, psum, psumsq,
        bn_gamma.reshape(-1, 1), bn_beta.reshape(-1, 1), c1, c2,
    )
    out = _sc_gather()(gidxT.T, wtsT.T, values)
    return out.reshape(bs, seq, d)


# speedup vs baseline: 1.0295x; 1.0295x over previous
import functools

import jax
import jax.numpy as jnp
from jax import lax
from jax.experimental import pallas as pl
from jax.experimental.pallas import tpu as pltpu
from jax.experimental.pallas import tpu_sc as plsc

_H = 4
_QD = 1024
_SKD = 512
_NSUB = 256
_K = 16
_DIM = 768
_SEQ = 2048
_HQD = _H * _QD

_TB = 256
_GRID = _SEQ // _TB

_STAIR = [(i, j) for i in range(_K) for j in range(_K) if (i + 1) * (j + 1) <= _K]

_NEG_INF = float("-inf")
_BIG_I = 1 << 30


_QTILE = 1024

def _mm_body(w_ref, x_ref, b_ref, q_ref, s_ref, ss_ref):
    xq = lax.dot_general(
        w_ref[...], x_ref[...],
        ((( 0,), (1,)), ((), ())),
        preferred_element_type=jnp.float32,
    ) + b_ref[...]
    q_ref[...] = xq
    s_ref[...] = jnp.sum(xq, axis=1, keepdims=True)
    ss_ref[...] = jnp.sum(xq * xq, axis=1, keepdims=True)


_matmul_stats = pl.pallas_call(
    _mm_body,
    grid=(_HQD // _QTILE,),
    in_specs=[
        pl.BlockSpec((_DIM, _QTILE), lambda i: (0, i)),
        pl.BlockSpec((_SEQ, _DIM), lambda i: (0, 0)),
        pl.BlockSpec((_QTILE, 1), lambda i: (i, 0)),
    ],
    out_specs=[
        pl.BlockSpec((_QTILE, _SEQ), lambda i: (i, 0)),
        pl.BlockSpec((_QTILE, 1), lambda i: (i, 0)),
        pl.BlockSpec((_QTILE, 1), lambda i: (i, 0)),
    ],
    out_shape=[
        jax.ShapeDtypeStruct((_HQD, _SEQ), jnp.float32),
        jax.ShapeDtypeStruct((_HQD, 1), jnp.float32),
        jax.ShapeDtypeStruct((_HQD, 1), jnp.float32),
    ],
)


def _topk16_vi(s_lo, s_hi):
    iota = lax.broadcasted_iota(jnp.int32, s_lo.shape, 0)
    work = s_lo
    vrows, irows = [], []
    for _ in range(_K):
        m = jnp.max(work, axis=0, keepdims=True)
        am = jnp.min(jnp.where(work == m, iota, _BIG_I), axis=0, keepdims=True)
        sel = iota == am
        v = jnp.min(jnp.where(sel, s_hi, jnp.inf), axis=0, keepdims=True)
        vrows.append(v)
        irows.append(am)
        work = jnp.where(sel, _NEG_INF, work)
    return jnp.concatenate(vrows, axis=0), jnp.concatenate(irows, axis=0)


def _topk16_payload(s, gi):
    iota = lax.broadcasted_iota(jnp.int32, s.shape, 0)
    work = s
    vrows, grows = [], []
    for _ in range(_K):
        m = jnp.max(work, axis=0, keepdims=True)
        am = jnp.min(jnp.where(work == m, iota, _BIG_I), axis=0, keepdims=True)
        sel = iota == am
        g = jnp.min(jnp.where(sel, gi, _BIG_I), axis=0, keepdims=True)
        vrows.append(m)
        grows.append(g)
        work = jnp.where(sel, _NEG_INF, work)
    return jnp.concatenate(vrows, axis=0), jnp.concatenate(grows, axis=0)


def _st_body(q_ref, s_ref, ss_ref, g_ref, b_ref, c1_ref, c2_ref, w_ref, gi_ref):
    ntok = jnp.float32(_SEQ)
    mean = s_ref[...] / ntok
    msq = ss_ref[...] / ntok
    var = msq - mean * mean
    scale = g_ref[...] * lax.rsqrt(var + 1e-5)
    shift = b_ref[...] - mean * scale
    qn = q_ref[...] * scale + shift

    for h in range(_H):
        q1 = qn[h * _QD : h * _QD + _SKD, :]
        q2 = qn[h * _QD + _SKD : (h + 1) * _QD, :]
        s1_lo = jnp.dot(c1_ref[h], q1, preferred_element_type=jnp.float32)
        s2_lo = jnp.dot(c2_ref[h], q2, preferred_element_type=jnp.float32)
        s1_hi = jnp.dot(c1_ref[h], q1, preferred_element_type=jnp.float32,
                        precision=lax.Precision.HIGHEST)
        s2_hi = jnp.dot(c2_ref[h], q2, preferred_element_type=jnp.float32,
                        precision=lax.Precision.HIGHEST)
        v1, i1 = _topk16_vi(s1_lo, s1_hi)
        v2, i2 = _topk16_vi(s2_lo, s2_hi)
        cand_rows, gi_rows = [], []
        for i in range(_K):
            cnt = sum(1 for ii, _ in _STAIR if ii == i)
            cand_rows.append(v1[i : i + 1, :] + v2[:cnt, :])
            gi_rows.append(i1[i : i + 1, :] * _NSUB + i2[:cnt, :])
        cand = jnp.concatenate(cand_rows, axis=0)
        gcand = jnp.concatenate(gi_rows, axis=0)
        tv, tg = _topk16_payload(cand, gcand)
        m = jnp.max(tv, axis=0, keepdims=True)
        e = jnp.exp(tv - m)
        wts = e / jnp.sum(e, axis=0, keepdims=True)
        w_ref[h * _K : (h + 1) * _K, :] = wts
        gi_ref[h * _K : (h + 1) * _K, :] = tg


_scores_topk = pl.pallas_call(
    _st_body,
    grid=(_GRID,),
    in_specs=[
        pl.BlockSpec((_HQD, _TB), lambda i: (0, i)),
        pl.BlockSpec((_HQD, 1), lambda i: (0, 0)),
        pl.BlockSpec((_HQD, 1), lambda i: (0, 0)),
        pl.BlockSpec((_HQD, 1), lambda i: (0, 0)),
        pl.BlockSpec((_HQD, 1), lambda i: (0, 0)),
        pl.BlockSpec((_H, _NSUB, _SKD), lambda i: (0, 0, 0)),
        pl.BlockSpec((_H, _NSUB, _SKD), lambda i: (0, 0, 0)),
    ],
    out_specs=[
        pl.BlockSpec((_H * _K, _TB), lambda i: (0, i)),
        pl.BlockSpec((_H * _K, _TB), lambda i: (0, i)),
    ],
    out_shape=[
        jax.ShapeDtypeStruct((_H * _K, _SEQ), jnp.float32),
        jax.ShapeDtypeStruct((_H * _K, _SEQ), jnp.int32),
    ],
)


_NC = 2
_NS = 16
_NW = _NC * _NS
_TPW = _SEQ // _NW
_R = _H * _K
_CG = 8
_NCG = _DIM // (16 * _CG)


def _sc_body(idx_hbm, w_hbm, values_hbm, out_hbm, idx_v, w_v, rows_v, outb_v,
             sem0, sem1, semo0, semo1):
    wid = lax.axis_index("s") * _NC + lax.axis_index("c")
    base = wid * _TPW
    pltpu.sync_copy(idx_hbm.at[pl.ds(base, _TPW)], idx_v)
    pltpu.sync_copy(w_hbm.at[pl.ds(base, _TPW)], w_v)
    sems = (sem0, sem1)
    semos = (semo0, semo1)

    def _issue(t, buf):
        pltpu.async_copy(values_hbm.at[idx_v.at[t]], rows_v.at[buf], sems[buf])

    _issue(0, 0)
    _issue(1, 1)

    def _token(t, buf):
        pltpu.make_async_copy(
            values_hbm.at[idx_v.at[t]], rows_v.at[buf], sems[buf]
        ).wait()

        @pl.when(t >= 2)
        def _():
            pltpu.make_async_copy(
                outb_v.at[buf], out_hbm.at[base + t - 2], semos[buf]
            ).wait()

        for cg in range(_NCG):
            def rgbody(rg, accs, _cg=cg):
                wvec = w_v[t, pl.ds(rg * 16, 16)]
                new = list(accs)
                for j in range(16):
                    wsc = wvec[j]
                    r = rg * 16 + j
                    for k in range(_CG):
                        new[k] = new[k] + (
                            rows_v[buf, r, pl.ds((_cg * _CG + k) * 16, 16)] * wsc
                        )
                return tuple(new)
            accs = lax.fori_loop(
                0, _R // 16, rgbody,
                tuple(jnp.zeros((16,), jnp.float32) for _ in range(_CG)),
            )
            for j in range(_CG):
                outb_v[buf, pl.ds((cg * _CG + j) * 16, 16)] = accs[j]
        pltpu.async_copy(outb_v.at[buf], out_hbm.at[base + t], semos[buf])

        @pl.when(t + 2 < _TPW)
        def _():
            _issue(t + 2, buf)

    def _outer(ti, carry):
        _token(ti * 2, 0)
        _token(ti * 2 + 1, 1)
        return carry

    lax.fori_loop(0, _TPW // 2, _outer, jnp.int32(0))
    pltpu.make_async_copy(
        outb_v.at[0], out_hbm.at[base + _TPW - 2], semo0).wait()
    pltpu.make_async_copy(
        outb_v.at[1], out_hbm.at[base + _TPW - 1], semo1).wait()


@functools.cache
def _sc_gather():
    return functools.partial(
        pl.kernel,
        mesh=plsc.VectorSubcoreMesh(core_axis_name="c", subcore_axis_name="s"),
        out_type=jax.ShapeDtypeStruct((_SEQ, _DIM), jnp.float32),
        scratch_types=[
            pltpu.VMEM((_TPW, _R), jnp.int32),
            pltpu.VMEM((_TPW, _R), jnp.float32),
            pltpu.VMEM((2, _R, _DIM), jnp.float32),
            pltpu.VMEM((2, _DIM), jnp.float32),
            pltpu.SemaphoreType.DMA,
            pltpu.SemaphoreType.DMA,
            pltpu.SemaphoreType.DMA,
            pltpu.SemaphoreType.DMA,
        ],
    )(_sc_body)


def kernel(x, W_q, b_q, bn_gamma, bn_beta, c1, c2, values):
    bs, seq, d = x.shape
    x_flat = x.reshape(-1, d)
    qT, psum, psumsq = _matmul_stats(W_q, x_flat, b_q.reshape(-1, 1))
    wtsT, gidxT = _scores_topk(
        qT, psum, psumsq, bn_gamma.reshape(-1, 1), bn_beta.reshape(-1, 1),
        c1, c2,
    )
    out = _sc_gather()(gidxT.T, wtsT.T, values)
    return out.reshape(bs, seq, d)
